# baseline (speedup 1.0000x reference)
import functools

import jax
import jax.numpy as jnp
from jax import lax
from jax.experimental import pallas as pl
from jax.experimental.pallas import tpu as pltpu
from jax.experimental.pallas import tpu_sc as plsc

N = 10000
E = 320000
IN = 128
H = 128
OUT = 128
G = 16

NTILES = 32
CHUNK = 128
NCH = 80
NRP = 8
NPH = NCH // NRP
E_PAD = NTILES * NCH * CHUNK
WAIT_WORDS = CHUNK
N_PAD = 10112
DUMMY = N_PAD - 1
ZR = N_PAD // 16

BN = 2000
BP = 1000
NB = N // BP


def _make_segsum(D, gather):
    scratch = [
        pltpu.VMEM((NCH, CHUNK), jnp.int32),
        pltpu.VMEM((NCH, CHUNK), jnp.int32),
        pltpu.VMEM((CHUNK, D), jnp.float32),
        pltpu.VMEM_SHARED((N_PAD, D), jnp.float32),
        pltpu.SemaphoreType.DMA,
    ]

    @functools.partial(
        pl.kernel,
        out_type=jax.ShapeDtypeStruct((2, N_PAD, D), jnp.float32),
        mesh=plsc.VectorSubcoreMesh(core_axis_name="c", subcore_axis_name="s"),
        scratch_types=scratch,
    )
    def segsum(h_hbm, srcm_hbm, dstm_hbm, zeros_hbm, out_hbm,
               idx_s, idx_d, rows, acc, sem):
        cid = lax.axis_index("c")
        sid = lax.axis_index("s")
        wid = cid * 16 + sid
        pltpu.sync_copy(zeros_hbm.at[pl.ds(sid * ZR, ZR)],
                        acc.at[pl.ds(sid * ZR, ZR)])
        pltpu.sync_copy(srcm_hbm.at[wid], idx_s)
        pltpu.sync_copy(dstm_hbm.at[wid], idx_d)
        if not gather:
            pltpu.sync_copy(h_hbm, rows)
        plsc.subcore_barrier()

        def step(j, carry):
            if gather:
                pltpu.async_copy(h_hbm.at[idx_s.at[j]], rows, sem).wait()
            pltpu.sync_copy(rows, acc.at[idx_d.at[j]], add=True)
            return carry

        lax.fori_loop(0, NCH, step, 0)
        plsc.subcore_barrier()
        pltpu.sync_copy(acc.at[pl.ds(sid * ZR, ZR)],
                        out_hbm.at[cid].at[pl.ds(sid * ZR, ZR)])

    return segsum


@functools.lru_cache(maxsize=None)
def _segsum(D, gather):
    return _make_segsum(D, gather)


def _ln_relu(z, gam, bet):
    m = jnp.mean(z, axis=-1, keepdims=True)
    v = jnp.mean((z - m) * (z - m), axis=-1, keepdims=True)
    return jnp.maximum((z - m) * lax.rsqrt(v + 1e-5) * gam + bet, 0.0)


def _embed_body(x_ref, w_ref, b_ref, g_ref, be_ref, o_ref):
    z = jnp.dot(x_ref[...], w_ref[...],
                preferred_element_type=jnp.float32) + b_ref[...]
    o_ref[...] = _ln_relu(z, g_ref[...], be_ref[...])


def _embed(x, w, b, g, be):
    vec = pl.BlockSpec((1, H), lambda i: (0, 0))
    return pl.pallas_call(
        _embed_body,
        out_shape=jax.ShapeDtypeStruct((N, H), jnp.float32),
        grid=(N // BN,),
        in_specs=[pl.BlockSpec((BN, IN), lambda i: (i, 0)),
                  pl.BlockSpec((IN, H), lambda i: (0, 0)), vec, vec, vec],
        out_specs=pl.BlockSpec((BN, H), lambda i: (i, 0)),
    )(x, w, b, g, be)


def _dense_body(h_ref, p_ref, c_ref, wt_ref, wb_ref,
                b_ref, g_ref, be_ref, o_ref):
    cnt = c_ref[0] + c_ref[1]
    inv = 1.0 / jnp.maximum(cnt[:, :1], 1.0)
    mean = (p_ref[0] + p_ref[1]) * inv
    z = (jnp.dot(h_ref[...], wt_ref[...], preferred_element_type=jnp.float32)
         + jnp.dot(mean, wb_ref[...], preferred_element_type=jnp.float32)
         + b_ref[...])
    o_ref[...] = _ln_relu(z, g_ref[...], be_ref[...])


def _dense(h, p, c, wt, wb, b, g, be):
    row = pl.BlockSpec((BN, H), lambda i: (i, 0))
    part = pl.BlockSpec((2, BN, H), lambda i: (0, i, 0))
    mat = pl.BlockSpec((H, H), lambda i: (0, 0))
    vec = pl.BlockSpec((1, H), lambda i: (0, 0))
    return pl.pallas_call(
        _dense_body,
        out_shape=jax.ShapeDtypeStruct((N, H), jnp.float32),
        grid=(N // BN,),
        in_specs=[row, part, part, mat, mat, vec, vec, vec],
        out_specs=row,
    )(h, p, c, wt, wb, b, g, be)


def _pool_body(ne_ref, bt_ref, mean_ref, max_ref, sum_s, cnt_s, max_s):
    i = pl.program_id(0)

    @pl.when(i == 0)
    def _init():
        sum_s[...] = jnp.zeros_like(sum_s)
        cnt_s[...] = jnp.zeros_like(cnt_s)
        max_s[...] = jnp.full_like(max_s, -jnp.inf)

    rows = ne_ref[...]
    b = bt_ref[...]
    oh = (b == lax.broadcasted_iota(jnp.int32, (BP, G), 1)
          ).astype(jnp.float32)
    dn = (((0,), (0,)), ((), ()))
    sum_s[...] += lax.dot_general(oh, rows, dn,
                                  preferred_element_type=jnp.float32)
    cnt_s[...] += lax.dot_general(oh, jnp.ones_like(rows), dn,
                                  preferred_element_type=jnp.float32)
    neg = jnp.full_like(rows, -jnp.inf)
    for gid in range(G):
        gm = jnp.max(jnp.where(b == gid, rows, neg), axis=0, keepdims=True)
        max_s[pl.ds(gid, 1), :] = jnp.maximum(max_s[pl.ds(gid, 1), :], gm)

    @pl.when(i == NB - 1)
    def _fin():
        mean_ref[...] = sum_s[...] / jnp.maximum(cnt_s[...], 1.0)
        max_ref[...] = max_s[...]


def _pool(ne, bt):
    out = jax.ShapeDtypeStruct((G, H), jnp.float32)
    return pl.pallas_call(
        _pool_body,
        out_shape=(out, out),
        grid=(NB,),
        in_specs=[pl.BlockSpec((BP, H), lambda i: (i, 0)),
                  pl.BlockSpec((BP, 1), lambda i: (i, 0))],
        out_specs=(pl.BlockSpec((G, H), lambda i: (0, 0)),
                   pl.BlockSpec((G, H), lambda i: (0, 0))),
        scratch_shapes=[pltpu.VMEM((G, H), jnp.float32),
                        pltpu.VMEM((G, H), jnp.float32),
                        pltpu.VMEM((G, H), jnp.float32)],
    )(ne, bt)


def kernel(x, W_emb, b_emb, g0, be0, W1, b1, g1, be1, W2, b2, g2, be2,
           W3, b3, g3, be3, edge_index, batch):
    src = edge_index[0]
    dst = edge_index[1]
    pad = E_PAD - E
    srcm = jnp.concatenate(
        [src, jnp.zeros((pad,), jnp.int32)]).reshape(NTILES, NCH, CHUNK)
    dstm = jnp.concatenate(
        [dst, jnp.full((pad,), DUMMY, jnp.int32)]).reshape(NTILES, NCH, CHUNK)
    zeros_h = jnp.zeros((N_PAD, H), jnp.float32)
    ones_c = jnp.ones((CHUNK, H), jnp.float32)

    cnt = _segsum(H, False)(ones_c, srcm, dstm, zeros_h)

    r = lambda v: v.reshape(1, H)
    h = _embed(x, W_emb, r(b_emb), r(g0), r(be0))
    for (W, b, gam, bet) in ((W1, b1, g1, be1), (W2, b2, g2, be2),
                             (W3, b3, g3, be3)):
        p = _segsum(H, True)(h, srcm, dstm, zeros_h)
        h = _dense(h, p, cnt, W[:H], W[H:], r(b), r(gam), r(bet))

    node_embed = h
    h_mean, h_max = _pool(h, batch.reshape(N, 1))
    graph_embed = jnp.concatenate([h_mean, h_max], axis=-1)
    return (node_embed, graph_embed)


# speedup vs baseline: 1.1507x; 1.1507x over previous
import functools

import jax
import jax.numpy as jnp
from jax import lax
from jax.experimental import pallas as pl
from jax.experimental.pallas import tpu as pltpu
from jax.experimental.pallas import tpu_sc as plsc

N = 10000
E = 320000
IN = 128
H = 128
OUT = 128
G = 16

NTILES = 32
CHUNK = 128
NCH = 80
NRP = 8
NPH = NCH // NRP
E_PAD = NTILES * NCH * CHUNK
WAIT_WORDS = CHUNK
N_PAD = 10112
DUMMY = N_PAD - 1
ZR = N_PAD // 16

BN = 2000
BP = 512
N_POOL = 10240
NB = N_POOL // BP


def _make_segsum(D, gather):
    scratch = [
        pltpu.VMEM((NCH, CHUNK), jnp.int32),
        pltpu.VMEM((NCH, CHUNK), jnp.int32),
        pltpu.VMEM((CHUNK, D), jnp.float32),
        pltpu.VMEM_SHARED((N_PAD, D), jnp.float32),
        pltpu.SemaphoreType.DMA,
    ]

    @functools.partial(
        pl.kernel,
        out_type=jax.ShapeDtypeStruct((2, N_PAD, D), jnp.float32),
        mesh=plsc.VectorSubcoreMesh(core_axis_name="c", subcore_axis_name="s"),
        scratch_types=scratch,
    )
    def segsum(h_hbm, srcm_hbm, dstm_hbm, zeros_hbm, out_hbm,
               idx_s, idx_d, rows, acc, sem):
        cid = lax.axis_index("c")
        sid = lax.axis_index("s")
        wid = cid * 16 + sid
        pltpu.sync_copy(zeros_hbm.at[pl.ds(sid * ZR, ZR)],
                        acc.at[pl.ds(sid * ZR, ZR)])
        pltpu.sync_copy(srcm_hbm.at[wid], idx_s)
        pltpu.sync_copy(dstm_hbm.at[wid], idx_d)
        if not gather:
            pltpu.sync_copy(h_hbm, rows)
        plsc.subcore_barrier()

        def step(j, carry):
            if gather:
                pltpu.async_copy(h_hbm.at[idx_s.at[j]], rows, sem).wait()
            pltpu.sync_copy(rows, acc.at[idx_d.at[j]], add=True)
            return carry

        lax.fori_loop(0, NCH, step, 0)
        plsc.subcore_barrier()
        pltpu.sync_copy(acc.at[pl.ds(sid * ZR, ZR)],
                        out_hbm.at[cid].at[pl.ds(sid * ZR, ZR)])

    return segsum


@functools.lru_cache(maxsize=None)
def _segsum(D, gather):
    return _make_segsum(D, gather)


def _ln_relu(z, gam, bet):
    m = jnp.mean(z, axis=-1, keepdims=True)
    v = jnp.mean((z - m) * (z - m), axis=-1, keepdims=True)
    return jnp.maximum((z - m) * lax.rsqrt(v + 1e-5) * gam + bet, 0.0)


def _embed_body(x_ref, w_ref, b_ref, g_ref, be_ref, o_ref):
    z = jnp.dot(x_ref[...], w_ref[...],
                preferred_element_type=jnp.float32) + b_ref[...]
    o_ref[...] = _ln_relu(z, g_ref[...], be_ref[...])


def _embed(x, w, b, g, be):
    vec = pl.BlockSpec((1, H), lambda i: (0, 0))
    return pl.pallas_call(
        _embed_body,
        out_shape=jax.ShapeDtypeStruct((N, H), jnp.float32),
        grid=(N // BN,),
        in_specs=[pl.BlockSpec((BN, IN), lambda i: (i, 0)),
                  pl.BlockSpec((IN, H), lambda i: (0, 0)), vec, vec, vec],
        out_specs=pl.BlockSpec((BN, H), lambda i: (i, 0)),
    )(x, w, b, g, be)


def _dense_body(h_ref, p0_ref, p1_ref, c0_ref, c1_ref, wt_ref, wb_ref,
                b_ref, g_ref, be_ref, o_ref):
    cnt = c0_ref[...] + c1_ref[...]
    inv = 1.0 / jnp.maximum(cnt[:, :1], 1.0)
    mean = (p0_ref[...] + p1_ref[...]) * inv
    z = (jnp.dot(h_ref[...], wt_ref[...], preferred_element_type=jnp.float32)
         + jnp.dot(mean, wb_ref[...], preferred_element_type=jnp.float32)
         + b_ref[...])
    o_ref[...] = _ln_relu(z, g_ref[...], be_ref[...])


def _dense(h, p0, p1, c0, c1, wt, wb, b, g, be):
    row = pl.BlockSpec((BN, H), lambda i: (i, 0))
    cntspec = pl.BlockSpec((BN, 16), lambda i: (i, 0))
    mat = pl.BlockSpec((H, H), lambda i: (0, 0))
    vec = pl.BlockSpec((1, H), lambda i: (0, 0))
    return pl.pallas_call(
        _dense_body,
        out_shape=jax.ShapeDtypeStruct((N, H), jnp.float32),
        grid=(N // BN,),
        in_specs=[row, row, row, cntspec, cntspec, mat, mat, vec, vec, vec],
        out_specs=row,
    )(h, p0, p1, c0, c1, wt, wb, b, g, be)


def _pool_body(ne_ref, bt_ref, mean_ref, max_ref, sum_s, cnt_s, max_s):
    i = pl.program_id(0)

    @pl.when(i == 0)
    def _init():
        sum_s[...] = jnp.zeros_like(sum_s)
        cnt_s[...] = jnp.zeros_like(cnt_s)
        max_s[...] = jnp.full_like(max_s, -jnp.inf)

    rows = ne_ref[...]
    b = bt_ref[...]
    oh = (b == lax.broadcasted_iota(jnp.int32, (BP, G), 1)
          ).astype(jnp.float32)
    dn = (((0,), (0,)), ((), ()))
    sum_s[...] += lax.dot_general(oh, rows, dn,
                                  preferred_element_type=jnp.float32)
    cnt_s[...] += lax.dot_general(oh, jnp.ones_like(rows), dn,
                                  preferred_element_type=jnp.float32)
    neg = jnp.full_like(rows, -jnp.inf)
    for gid in range(G):
        gm = jnp.max(jnp.where(b == gid, rows, neg), axis=0, keepdims=True)
        max_s[pl.ds(gid, 1), :] = jnp.maximum(max_s[pl.ds(gid, 1), :], gm)

    @pl.when(i == NB - 1)
    def _fin():
        mean_ref[...] = sum_s[...] / jnp.maximum(cnt_s[...], 1.0)
        max_ref[...] = max_s[...]


def _pool(ne, bt):
    out = jax.ShapeDtypeStruct((G, H), jnp.float32)
    return pl.pallas_call(
        _pool_body,
        out_shape=(out, out),
        grid=(NB,),
        in_specs=[pl.BlockSpec((BP, H), lambda i: (i, 0)),
                  pl.BlockSpec((BP, 1), lambda i: (i, 0))],
        out_specs=(pl.BlockSpec((G, H), lambda i: (0, 0)),
                   pl.BlockSpec((G, H), lambda i: (0, 0))),
        scratch_shapes=[pltpu.VMEM((G, H), jnp.float32),
                        pltpu.VMEM((G, H), jnp.float32),
                        pltpu.VMEM((G, H), jnp.float32)],
    )(ne, bt)


def kernel(x, W_emb, b_emb, g0, be0, W1, b1, g1, be1, W2, b2, g2, be2,
           W3, b3, g3, be3, edge_index, batch):
    src = edge_index[0]
    dst = edge_index[1]
    pad = E_PAD - E
    srcm = jnp.concatenate(
        [src, jnp.zeros((pad,), jnp.int32)]).reshape(NTILES, NCH, CHUNK)
    dstm = jnp.concatenate(
        [dst, jnp.full((pad,), DUMMY, jnp.int32)]).reshape(NTILES, NCH, CHUNK)
    zeros_h = jnp.zeros((N_PAD, H), jnp.float32)
    ones_c = jnp.ones((CHUNK, H), jnp.float32)

    cnt = _segsum(H, False)(ones_c, srcm, dstm, zeros_h)
    c0 = cnt[0, :N, :16]
    c1 = cnt[1, :N, :16]

    r = lambda v: v.reshape(1, H)
    h = _embed(x, W_emb, r(b_emb), r(g0), r(be0))
    for (W, b, gam, bet) in ((W1, b1, g1, be1), (W2, b2, g2, be2),
                             (W3, b3, g3, be3)):
        p = _segsum(H, True)(h, srcm, dstm, zeros_h)
        h = _dense(h, p[0, :N], p[1, :N], c0, c1,
                   W[:H], W[H:], r(b), r(gam), r(bet))

    node_embed = h
    ne_p = jnp.concatenate([h, jnp.zeros((N_POOL - N, H), jnp.float32)])
    bt_p = jnp.concatenate(
        [batch, jnp.full((N_POOL - N,), G, jnp.int32)]).reshape(N_POOL, 1)
    h_mean, h_max = _pool(ne_p, bt_p)
    graph_embed = jnp.concatenate([h_mean, h_max], axis=-1)
    return (node_embed, graph_embed)


# speedup vs baseline: 1.6463x; 1.4306x over previous
import functools

import jax
import jax.numpy as jnp
from jax import lax
from jax.experimental import pallas as pl
from jax.experimental.pallas import tpu as pltpu
from jax.experimental.pallas import tpu_sc as plsc

N = 10000
E = 320000
IN = 128
H = 128
OUT = 128
G = 16

NTILES = 32
CHUNK = 128
NCH = 79
E_PAD = NTILES * NCH * CHUNK
WAIT_WORDS = CHUNK
N_PAD = 10112
DUMMY = N_PAD - 1
ZR = N_PAD // 16

BN = 2000
BP = 512
N_POOL = 10240
NB = N_POOL // BP


def _make_segsum(D, gather):
    scratch = [
        pltpu.VMEM((NCH, CHUNK), jnp.int32),
        pltpu.VMEM((NCH, CHUNK), jnp.int32),
        pltpu.VMEM((CHUNK, D), jnp.float32),
        pltpu.VMEM_SHARED((N_PAD, D), jnp.float32),
        pltpu.SemaphoreType.DMA,
    ]

    @functools.partial(
        pl.kernel,
        out_type=jax.ShapeDtypeStruct((2, N_PAD, D), jnp.float32),
        mesh=plsc.VectorSubcoreMesh(core_axis_name="c", subcore_axis_name="s"),
        scratch_types=scratch,
    )
    def segsum(h_hbm, srcm_hbm, dstm_hbm, zeros_hbm, out_hbm,
               idx_s, idx_d, rows, acc, sem):
        cid = lax.axis_index("c")
        sid = lax.axis_index("s")
        wid = cid * 16 + sid
        pltpu.sync_copy(zeros_hbm.at[pl.ds(sid * ZR, ZR)],
                        acc.at[pl.ds(sid * ZR, ZR)])
        pltpu.sync_copy(srcm_hbm.at[wid], idx_s)
        pltpu.sync_copy(dstm_hbm.at[wid], idx_d)
        if not gather:
            pltpu.sync_copy(h_hbm, rows)
        plsc.subcore_barrier()

        def step(j, carry):
            if gather:
                pltpu.async_copy(h_hbm.at[idx_s.at[j]], rows, sem).wait()
            pltpu.sync_copy(rows, acc.at[idx_d.at[j]], add=True)
            return carry

        lax.fori_loop(0, NCH, step, 0)
        plsc.subcore_barrier()
        pltpu.sync_copy(acc.at[pl.ds(sid * ZR, ZR)],
                        out_hbm.at[cid].at[pl.ds(sid * ZR, ZR)])

    return segsum


@functools.lru_cache(maxsize=None)
def _segsum(D, gather):
    return _make_segsum(D, gather)


def _ln_relu(z, gam, bet):
    m = jnp.mean(z, axis=-1, keepdims=True)
    v = jnp.mean((z - m) * (z - m), axis=-1, keepdims=True)
    return jnp.maximum((z - m) * lax.rsqrt(v + 1e-5) * gam + bet, 0.0)


def _embed_body(x_ref, w_ref, b_ref, g_ref, be_ref, o_ref):
    z = jnp.dot(x_ref[...], w_ref[...],
                preferred_element_type=jnp.float32) + b_ref[...]
    o_ref[...] = _ln_relu(z, g_ref[...], be_ref[...])


def _embed(x, w, b, g, be):
    vec = pl.BlockSpec((1, H), lambda i: (0, 0))
    return pl.pallas_call(
        _embed_body,
        out_shape=jax.ShapeDtypeStruct((N, H), jnp.float32),
        grid=(N // BN,),
        in_specs=[pl.BlockSpec((BN, IN), lambda i: (i, 0)),
                  pl.BlockSpec((IN, H), lambda i: (0, 0)), vec, vec, vec],
        out_specs=pl.BlockSpec((BN, H), lambda i: (i, 0)),
    )(x, w, b, g, be)


def _dense_body(h_ref, p0_ref, p1_ref, c0_ref, c1_ref, wt_ref, wb_ref,
                b_ref, g_ref, be_ref, o_ref):
    cnt = c0_ref[...] + c1_ref[...]
    inv = 1.0 / jnp.maximum(cnt[:, :1], 1.0)
    mean = (p0_ref[...] + p1_ref[...]) * inv
    z = (jnp.dot(h_ref[...], wt_ref[...], preferred_element_type=jnp.float32)
         + jnp.dot(mean, wb_ref[...], preferred_element_type=jnp.float32)
         + b_ref[...])
    o_ref[...] = _ln_relu(z, g_ref[...], be_ref[...])


def _dense(h, p0, p1, c0, c1, wt, wb, b, g, be):
    row = pl.BlockSpec((BN, H), lambda i: (i, 0))
    cntspec = pl.BlockSpec((BN, 16), lambda i: (i, 0))
    mat = pl.BlockSpec((H, H), lambda i: (0, 0))
    vec = pl.BlockSpec((1, H), lambda i: (0, 0))
    return pl.pallas_call(
        _dense_body,
        out_shape=jax.ShapeDtypeStruct((N, H), jnp.float32),
        grid=(N // BN,),
        in_specs=[row, row, row, cntspec, cntspec, mat, mat, vec, vec, vec],
        out_specs=row,
    )(h, p0, p1, c0, c1, wt, wb, b, g, be)


def _pool_body(ne_ref, bt_ref, mean_ref, max_ref, sum_s, cnt_s, max_s):
    i = pl.program_id(0)

    @pl.when(i == 0)
    def _init():
        sum_s[...] = jnp.zeros_like(sum_s)
        cnt_s[...] = jnp.zeros_like(cnt_s)
        max_s[...] = jnp.full_like(max_s, -jnp.inf)

    rows = ne_ref[...]
    b = bt_ref[...]
    oh = (b == lax.broadcasted_iota(jnp.int32, (BP, G), 1)
          ).astype(jnp.float32)
    dn = (((0,), (0,)), ((), ()))
    sum_s[...] += lax.dot_general(oh, rows, dn,
                                  preferred_element_type=jnp.float32)
    cnt_s[...] += lax.dot_general(oh, jnp.ones_like(rows), dn,
                                  preferred_element_type=jnp.float32)
    neg = jnp.full_like(rows, -jnp.inf)
    for gid in range(G):
        gm = jnp.max(jnp.where(b == gid, rows, neg), axis=0, keepdims=True)
        max_s[pl.ds(gid, 1), :] = jnp.maximum(max_s[pl.ds(gid, 1), :], gm)

    @pl.when(i == NB - 1)
    def _fin():
        mean_ref[...] = sum_s[...] / jnp.maximum(cnt_s[...], 1.0)
        max_ref[...] = max_s[...]


def _pool(ne, bt):
    out = jax.ShapeDtypeStruct((G, H), jnp.float32)
    return pl.pallas_call(
        _pool_body,
        out_shape=(out, out),
        grid=(NB,),
        in_specs=[pl.BlockSpec((BP, H), lambda i: (i, 0)),
                  pl.BlockSpec((BP, 1), lambda i: (i, 0))],
        out_specs=(pl.BlockSpec((G, H), lambda i: (0, 0)),
                   pl.BlockSpec((G, H), lambda i: (0, 0))),
        scratch_shapes=[pltpu.VMEM((G, H), jnp.float32),
                        pltpu.VMEM((G, H), jnp.float32),
                        pltpu.VMEM((G, H), jnp.float32)],
    )(ne, bt)


def kernel(x, W_emb, b_emb, g0, be0, W1, b1, g1, be1, W2, b2, g2, be2,
           W3, b3, g3, be3, edge_index, batch):
    src = edge_index[0]
    dst = edge_index[1]
    pad = E_PAD - E
    srcm = jnp.concatenate(
        [src, jnp.zeros((pad,), jnp.int32)]).reshape(NTILES, NCH, CHUNK)
    dst_pad = N + jnp.arange(pad, dtype=jnp.int32) % (N_PAD - N)
    dstm = jnp.concatenate([dst, dst_pad]).reshape(NTILES, NCH, CHUNK)
    zeros_h = jnp.zeros((N_PAD, H), jnp.float32)
    ones_c = jnp.ones((CHUNK, H), jnp.float32)

    cnt = _segsum(H, False)(ones_c, srcm, dstm, zeros_h)
    c0 = cnt[0, :N, :16]
    c1 = cnt[1, :N, :16]

    r = lambda v: v.reshape(1, H)
    h = _embed(x, W_emb, r(b_emb), r(g0), r(be0))
    for (W, b, gam, bet) in ((W1, b1, g1, be1), (W2, b2, g2, be2),
                             (W3, b3, g3, be3)):
        p = _segsum(H, True)(h, srcm, dstm, zeros_h)
        h = _dense(h, p[0, :N], p[1, :N], c0, c1,
                   W[:H], W[H:], r(b), r(gam), r(bet))

    node_embed = h
    ne_p = jnp.concatenate([h, jnp.zeros((N_POOL - N, H), jnp.float32)])
    bt_p = jnp.concatenate(
        [batch, jnp.full((N_POOL - N,), G, jnp.int32)]).reshape(N_POOL, 1)
    h_mean, h_max = _pool(ne_p, bt_p)
    graph_embed = jnp.concatenate([h_mean, h_max], axis=-1)
    return (node_embed, graph_embed)


# speedup vs baseline: 1.7549x; 1.0660x over previous
import functools

import jax
import jax.numpy as jnp
from jax import lax
from jax.experimental import pallas as pl
from jax.experimental.pallas import tpu as pltpu
from jax.experimental.pallas import tpu_sc as plsc

N = 10000
E = 320000
IN = 128
H = 128
OUT = 128
G = 16

NTILES = 32
CHUNK = 128
NCH0 = 102
NCH1 = 56
E_SC0 = 16 * NCH0 * CHUNK
E_SC1 = 16 * NCH1 * CHUNK
E_PAD = E_SC0 + E_SC1
N_PAD = 10112
DUMMY = N_PAD - 1
ZR = N_PAD // 16

BN = 2000
BP = 512
N_POOL = 10240
NB = N_POOL // BP


def _make_segsum(D, gather):
    scratch = [
        pltpu.VMEM((NCH0, CHUNK), jnp.int32),
        pltpu.VMEM((NCH0, CHUNK), jnp.int32),
        pltpu.VMEM((CHUNK, D), jnp.float32),
        pltpu.VMEM_SHARED((N_PAD, D), jnp.float32),
        pltpu.SemaphoreType.DMA,
    ]

    @functools.partial(
        pl.kernel,
        out_type=jax.ShapeDtypeStruct((2, N_PAD, D), jnp.float32),
        mesh=plsc.VectorSubcoreMesh(core_axis_name="c", subcore_axis_name="s"),
        scratch_types=scratch,
    )
    def segsum(h_hbm, srcm_hbm, dstm_hbm, zeros_hbm, out_hbm,
               idx_s, idx_d, rows, acc, sem):
        cid = lax.axis_index("c")
        sid = lax.axis_index("s")
        wid = cid * 16 + sid
        pltpu.sync_copy(zeros_hbm.at[pl.ds(sid * ZR, ZR)],
                        acc.at[pl.ds(sid * ZR, ZR)])
        pltpu.sync_copy(srcm_hbm.at[wid], idx_s)
        pltpu.sync_copy(dstm_hbm.at[wid], idx_d)
        if not gather:
            pltpu.sync_copy(h_hbm, rows)
        plsc.subcore_barrier()

        def step(j, carry):
            if gather:
                pltpu.async_copy(h_hbm.at[idx_s.at[j]], rows, sem).wait()
            pltpu.sync_copy(rows, acc.at[idx_d.at[j]], add=True)
            return carry

        @pl.when(cid == 0)
        def _():
            lax.fori_loop(0, NCH0, step, 0)

        @pl.when(cid == 1)
        def _():
            lax.fori_loop(0, NCH1, step, 0)

        plsc.subcore_barrier()
        pltpu.sync_copy(acc.at[pl.ds(sid * ZR, ZR)],
                        out_hbm.at[cid].at[pl.ds(sid * ZR, ZR)])

    return segsum


@functools.lru_cache(maxsize=None)
def _segsum(D, gather):
    return _make_segsum(D, gather)


def _ln_relu(z, gam, bet):
    m = jnp.mean(z, axis=-1, keepdims=True)
    v = jnp.mean((z - m) * (z - m), axis=-1, keepdims=True)
    return jnp.maximum((z - m) * lax.rsqrt(v + 1e-5) * gam + bet, 0.0)


def _embed_body(x_ref, w_ref, b_ref, g_ref, be_ref, o_ref):
    z = jnp.dot(x_ref[...], w_ref[...],
                preferred_element_type=jnp.float32) + b_ref[...]
    o_ref[...] = _ln_relu(z, g_ref[...], be_ref[...])


def _embed(x, w, b, g, be):
    vec = pl.BlockSpec((1, H), lambda i: (0, 0))
    return pl.pallas_call(
        _embed_body,
        out_shape=jax.ShapeDtypeStruct((N, H), jnp.float32),
        grid=(N // BN,),
        in_specs=[pl.BlockSpec((BN, IN), lambda i: (i, 0)),
                  pl.BlockSpec((IN, H), lambda i: (0, 0)), vec, vec, vec],
        out_specs=pl.BlockSpec((BN, H), lambda i: (i, 0)),
    )(x, w, b, g, be)


def _dense_body(h_ref, p0_ref, p1_ref, c0_ref, c1_ref, wt_ref, wb_ref,
                b_ref, g_ref, be_ref, o_ref):
    cnt = c0_ref[...] + c1_ref[...]
    inv = 1.0 / jnp.maximum(cnt[:, :1], 1.0)
    mean = (p0_ref[...] + p1_ref[...]) * inv
    z = (jnp.dot(h_ref[...], wt_ref[...], preferred_element_type=jnp.float32)
         + jnp.dot(mean, wb_ref[...], preferred_element_type=jnp.float32)
         + b_ref[...])
    o_ref[...] = _ln_relu(z, g_ref[...], be_ref[...])


def _dense(h, p0, p1, c0, c1, wt, wb, b, g, be):
    row = pl.BlockSpec((BN, H), lambda i: (i, 0))
    cntspec = pl.BlockSpec((BN, 16), lambda i: (i, 0))
    mat = pl.BlockSpec((H, H), lambda i: (0, 0))
    vec = pl.BlockSpec((1, H), lambda i: (0, 0))
    return pl.pallas_call(
        _dense_body,
        out_shape=jax.ShapeDtypeStruct((N, H), jnp.float32),
        grid=(N // BN,),
        in_specs=[row, row, row, cntspec, cntspec, mat, mat, vec, vec, vec],
        out_specs=row,
    )(h, p0, p1, c0, c1, wt, wb, b, g, be)


def _pool_body(ne_ref, bt_ref, mean_ref, max_ref, sum_s, cnt_s, max_s):
    i = pl.program_id(0)

    @pl.when(i == 0)
    def _init():
        sum_s[...] = jnp.zeros_like(sum_s)
        cnt_s[...] = jnp.zeros_like(cnt_s)
        max_s[...] = jnp.full_like(max_s, -jnp.inf)

    rows = ne_ref[...]
    b = bt_ref[...]
    oh = (b == lax.broadcasted_iota(jnp.int32, (BP, G), 1)
          ).astype(jnp.float32)
    dn = (((0,), (0,)), ((), ()))
    sum_s[...] += lax.dot_general(oh, rows, dn,
                                  preferred_element_type=jnp.float32)
    cnt_s[...] += lax.dot_general(oh, jnp.ones_like(rows), dn,
                                  preferred_element_type=jnp.float32)
    neg = jnp.full_like(rows, -jnp.inf)
    for gid in range(G):
        gm = jnp.max(jnp.where(b == gid, rows, neg), axis=0, keepdims=True)
        max_s[pl.ds(gid, 1), :] = jnp.maximum(max_s[pl.ds(gid, 1), :], gm)

    @pl.when(i == NB - 1)
    def _fin():
        mean_ref[...] = sum_s[...] / jnp.maximum(cnt_s[...], 1.0)
        max_ref[...] = max_s[...]


def _pool(ne, bt):
    out = jax.ShapeDtypeStruct((G, H), jnp.float32)
    return pl.pallas_call(
        _pool_body,
        out_shape=(out, out),
        grid=(NB,),
        in_specs=[pl.BlockSpec((BP, H), lambda i: (i, 0)),
                  pl.BlockSpec((BP, 1), lambda i: (i, 0))],
        out_specs=(pl.BlockSpec((G, H), lambda i: (0, 0)),
                   pl.BlockSpec((G, H), lambda i: (0, 0))),
        scratch_shapes=[pltpu.VMEM((G, H), jnp.float32),
                        pltpu.VMEM((G, H), jnp.float32),
                        pltpu.VMEM((G, H), jnp.float32)],
    )(ne, bt)


def kernel(x, W_emb, b_emb, g0, be0, W1, b1, g1, be1, W2, b2, g2, be2,
           W3, b3, g3, be3, edge_index, batch):
    src = edge_index[0]
    dst = edge_index[1]
    pad = E_PAD - E
    dst_pad = N + jnp.arange(pad, dtype=jnp.int32) % (N_PAD - N)
    src0 = src[:E_SC0].reshape(16, NCH0, CHUNK)
    dst0 = dst[:E_SC0].reshape(16, NCH0, CHUNK)
    grow = ((0, 0), (0, NCH0 - NCH1), (0, 0))
    src1 = jnp.pad(jnp.concatenate(
        [src[E_SC0:], jnp.zeros((pad,), jnp.int32)]).reshape(16, NCH1, CHUNK),
        grow)
    dst1 = jnp.pad(jnp.concatenate(
        [dst[E_SC0:], dst_pad]).reshape(16, NCH1, CHUNK),
        grow, constant_values=DUMMY)
    srcm = jnp.concatenate([src0, src1])
    dstm = jnp.concatenate([dst0, dst1])
    zeros_h = jnp.zeros((N_PAD, H), jnp.float32)
    ones_c = jnp.ones((CHUNK, H), jnp.float32)

    cnt = _segsum(H, False)(ones_c, srcm, dstm, zeros_h)
    c0 = cnt[0, :N, :16]
    c1 = cnt[1, :N, :16]

    r = lambda v: v.reshape(1, H)
    h = _embed(x, W_emb, r(b_emb), r(g0), r(be0))
    for (W, b, gam, bet) in ((W1, b1, g1, be1), (W2, b2, g2, be2),
                             (W3, b3, g3, be3)):
        p = _segsum(H, True)(h, srcm, dstm, zeros_h)
        h = _dense(h, p[0, :N], p[1, :N], c0, c1,
                   W[:H], W[H:], r(b), r(gam), r(bet))

    node_embed = h
    ne_p = jnp.concatenate([h, jnp.zeros((N_POOL - N, H), jnp.float32)])
    bt_p = jnp.concatenate(
        [batch, jnp.full((N_POOL - N,), G, jnp.int32)]).reshape(N_POOL, 1)
    h_mean, h_max = _pool(ne_p, bt_p)
    graph_embed = jnp.concatenate([h_mean, h_max], axis=-1)
    return (node_embed, graph_embed)


# speedup vs baseline: 1.7929x; 1.0217x over previous
import functools

import jax
import jax.numpy as jnp
from jax import lax
from jax.experimental import pallas as pl
from jax.experimental.pallas import tpu as pltpu
from jax.experimental.pallas import tpu_sc as plsc

N = 10000
E = 320000
IN = 128
H = 128
OUT = 128
G = 16

NTILES = 32
CHUNK = 128
NCH0 = 124
NCH1 = 34
E_SC0 = 16 * NCH0 * CHUNK
E_SC1 = 16 * NCH1 * CHUNK
E_PAD = E_SC0 + E_SC1
NCHB = E_PAD // (NTILES * CHUNK)
N_PAD = 10112
DUMMY = N_PAD - 1
ZR = N_PAD // 16

BN = 2000
BP = 512
N_POOL = 10240
NB = N_POOL // BP


def _make_segsum(D, gather):
    nch0, nch1 = (NCH0, NCH1) if gather else (NCHB, NCHB)
    scratch = [
        pltpu.VMEM((nch0 if gather else 1, CHUNK), jnp.int32),
        pltpu.VMEM((nch0, CHUNK), jnp.int32),
        pltpu.VMEM((CHUNK, D), jnp.float32),
        pltpu.VMEM_SHARED((N_PAD, D), jnp.float32),
        pltpu.SemaphoreType.DMA,
    ]

    @functools.partial(
        pl.kernel,
        out_type=jax.ShapeDtypeStruct((2, N_PAD, D), jnp.float32),
        mesh=plsc.VectorSubcoreMesh(core_axis_name="c", subcore_axis_name="s"),
        scratch_types=scratch,
    )
    def segsum(h_hbm, srcm_hbm, dstm_hbm, zeros_hbm, out_hbm,
               idx_s, idx_d, rows, acc, sem):
        cid = lax.axis_index("c")
        sid = lax.axis_index("s")
        wid = cid * 16 + sid
        pltpu.sync_copy(zeros_hbm.at[pl.ds(sid * ZR, ZR)],
                        acc.at[pl.ds(sid * ZR, ZR)])
        pltpu.sync_copy(dstm_hbm.at[wid], idx_d)
        if gather:
            pltpu.sync_copy(srcm_hbm.at[wid], idx_s)
        else:
            pltpu.sync_copy(h_hbm, rows)
        plsc.subcore_barrier()

        def step(j, carry):
            if gather:
                pltpu.async_copy(h_hbm.at[idx_s.at[j]], rows, sem).wait()
            pltpu.sync_copy(rows, acc.at[idx_d.at[j]], add=True)
            return carry

        if nch0 == nch1:
            lax.fori_loop(0, nch0, step, 0)
        else:
            @pl.when(cid == 0)
            def _():
                lax.fori_loop(0, nch0, step, 0)

            @pl.when(cid == 1)
            def _():
                lax.fori_loop(0, nch1, step, 0)

        plsc.subcore_barrier()
        pltpu.sync_copy(acc.at[pl.ds(sid * ZR, ZR)],
                        out_hbm.at[cid].at[pl.ds(sid * ZR, ZR)])

    return segsum


@functools.lru_cache(maxsize=None)
def _segsum(D, gather):
    return _make_segsum(D, gather)


def _ln_relu(z, gam, bet):
    m = jnp.mean(z, axis=-1, keepdims=True)
    v = jnp.mean((z - m) * (z - m), axis=-1, keepdims=True)
    return jnp.maximum((z - m) * lax.rsqrt(v + 1e-5) * gam + bet, 0.0)


def _embed_body(x_ref, w_ref, b_ref, g_ref, be_ref, o_ref):
    z = jnp.dot(x_ref[...], w_ref[...],
                preferred_element_type=jnp.float32) + b_ref[...]
    o_ref[...] = _ln_relu(z, g_ref[...], be_ref[...])


def _embed(x, w, b, g, be):
    vec = pl.BlockSpec((1, H), lambda i: (0, 0))
    return pl.pallas_call(
        _embed_body,
        out_shape=jax.ShapeDtypeStruct((N, H), jnp.float32),
        grid=(N // BN,),
        in_specs=[pl.BlockSpec((BN, IN), lambda i: (i, 0)),
                  pl.BlockSpec((IN, H), lambda i: (0, 0)), vec, vec, vec],
        out_specs=pl.BlockSpec((BN, H), lambda i: (i, 0)),
    )(x, w, b, g, be)


def _dense_body(h_ref, p0_ref, p1_ref, c0_ref, c1_ref, wt_ref, wb_ref,
                b_ref, g_ref, be_ref, o_ref):
    cnt = c0_ref[...] + c1_ref[...]
    inv = 1.0 / jnp.maximum(cnt[:, :1], 1.0)
    mean = (p0_ref[...] + p1_ref[...]) * inv
    z = (jnp.dot(h_ref[...], wt_ref[...], preferred_element_type=jnp.float32)
         + jnp.dot(mean, wb_ref[...], preferred_element_type=jnp.float32)
         + b_ref[...])
    o_ref[...] = _ln_relu(z, g_ref[...], be_ref[...])


def _dense(h, p0, p1, c0, c1, wt, wb, b, g, be):
    row = pl.BlockSpec((BN, H), lambda i: (i, 0))
    cntspec = pl.BlockSpec((BN, 16), lambda i: (i, 0))
    mat = pl.BlockSpec((H, H), lambda i: (0, 0))
    vec = pl.BlockSpec((1, H), lambda i: (0, 0))
    return pl.pallas_call(
        _dense_body,
        out_shape=jax.ShapeDtypeStruct((N, H), jnp.float32),
        grid=(N // BN,),
        in_specs=[row, row, row, cntspec, cntspec, mat, mat, vec, vec, vec],
        out_specs=row,
    )(h, p0, p1, c0, c1, wt, wb, b, g, be)


def _pool_body(ne_ref, bt_ref, mean_ref, max_ref, sum_s, cnt_s, max_s):
    i = pl.program_id(0)

    @pl.when(i == 0)
    def _init():
        sum_s[...] = jnp.zeros_like(sum_s)
        cnt_s[...] = jnp.zeros_like(cnt_s)
        max_s[...] = jnp.full_like(max_s, -jnp.inf)

    rows = ne_ref[...]
    b = bt_ref[...]
    oh = (b == lax.broadcasted_iota(jnp.int32, (BP, G), 1)
          ).astype(jnp.float32)
    dn = (((0,), (0,)), ((), ()))
    sum_s[...] += lax.dot_general(oh, rows, dn,
                                  preferred_element_type=jnp.float32)
    cnt_s[...] += lax.dot_general(oh, jnp.ones_like(rows), dn,
                                  preferred_element_type=jnp.float32)
    neg = jnp.full_like(rows, -jnp.inf)
    for gid in range(G):
        gm = jnp.max(jnp.where(b == gid, rows, neg), axis=0, keepdims=True)
        max_s[pl.ds(gid, 1), :] = jnp.maximum(max_s[pl.ds(gid, 1), :], gm)

    @pl.when(i == NB - 1)
    def _fin():
        mean_ref[...] = sum_s[...] / jnp.maximum(cnt_s[...], 1.0)
        max_ref[...] = max_s[...]


def _pool(ne, bt):
    out = jax.ShapeDtypeStruct((G, H), jnp.float32)
    return pl.pallas_call(
        _pool_body,
        out_shape=(out, out),
        grid=(NB,),
        in_specs=[pl.BlockSpec((BP, H), lambda i: (i, 0)),
                  pl.BlockSpec((BP, 1), lambda i: (i, 0))],
        out_specs=(pl.BlockSpec((G, H), lambda i: (0, 0)),
                   pl.BlockSpec((G, H), lambda i: (0, 0))),
        scratch_shapes=[pltpu.VMEM((G, H), jnp.float32),
                        pltpu.VMEM((G, H), jnp.float32),
                        pltpu.VMEM((G, H), jnp.float32)],
    )(ne, bt)


def kernel(x, W_emb, b_emb, g0, be0, W1, b1, g1, be1, W2, b2, g2, be2,
           W3, b3, g3, be3, edge_index, batch):
    src = edge_index[0]
    dst = edge_index[1]
    pad = E_PAD - E
    dst_pad = N + jnp.arange(pad, dtype=jnp.int32) % (N_PAD - N)
    src0 = src[:E_SC0].reshape(16, NCH0, CHUNK)
    dst0 = dst[:E_SC0].reshape(16, NCH0, CHUNK)
    grow = ((0, 0), (0, NCH0 - NCH1), (0, 0))
    src1 = jnp.pad(jnp.concatenate(
        [src[E_SC0:], jnp.zeros((pad,), jnp.int32)]).reshape(16, NCH1, CHUNK),
        grow)
    dst1 = jnp.pad(jnp.concatenate(
        [dst[E_SC0:], dst_pad]).reshape(16, NCH1, CHUNK),
        grow, constant_values=DUMMY)
    srcm = jnp.concatenate([src0, src1])
    dstm = jnp.concatenate([dst0, dst1])
    dstm_bal = jnp.concatenate([dst, dst_pad]).reshape(NTILES, NCHB, CHUNK)
    zeros_h = jnp.zeros((N_PAD, H), jnp.float32)
    ones_c = jnp.ones((CHUNK, H), jnp.float32)

    cnt = _segsum(H, False)(ones_c, srcm, dstm_bal, zeros_h)
    c0 = cnt[0, :N, :16]
    c1 = cnt[1, :N, :16]

    r = lambda v: v.reshape(1, H)
    h = _embed(x, W_emb, r(b_emb), r(g0), r(be0))
    for (W, b, gam, bet) in ((W1, b1, g1, be1), (W2, b2, g2, be2),
                             (W3, b3, g3, be3)):
        p = _segsum(H, True)(h, srcm, dstm, zeros_h)
        h = _dense(h, p[0, :N], p[1, :N], c0, c1,
                   W[:H], W[H:], r(b), r(gam), r(bet))

    node_embed = h
    ne_p = jnp.concatenate([h, jnp.zeros((N_POOL - N, H), jnp.float32)])
    bt_p = jnp.concatenate(
        [batch, jnp.full((N_POOL - N,), G, jnp.int32)]).reshape(N_POOL, 1)
    h_mean, h_max = _pool(ne_p, bt_p)
    graph_embed = jnp.concatenate([h_mean, h_max], axis=-1)
    return (node_embed, graph_embed)
